# baseline (speedup 1.0000x reference)
import functools

import jax
import jax.numpy as jnp
from jax import lax
from jax.experimental import pallas as pl
from jax.experimental.pallas import tpu as pltpu
from jax.experimental.pallas import tpu_sc as plsc

N_NODES = 10000
N_EDGES = 320000
IN_FEATS = 128
HIDDEN = 64
OUT_FEATS = 32

NC = 2
NS = 16
CHUNK = 128
CHUNKS_PER_W = 79
E_PER_W = CHUNKS_PER_W * CHUNK
E_PAD = NC * NS * E_PER_W
N_PAD = 10240
ROWS_PER_S = N_PAD // NS
CNT_W = 8


NBUF = 3
C_SPLIT = (53, 105)
C_MAX = max(C_SPLIT)
assert sum(C_SPLIT) == 2 * CHUNKS_PER_W


def _sc_edge_agg(D, with_count):
    mesh = plsc.VectorSubcoreMesh(
        core_axis_name="c", subcore_axis_name="s",
        num_cores=NC, num_subcores=NS)

    out_type = [jax.ShapeDtypeStruct((NC, N_PAD, D), jnp.float32)]
    scratch = [
        pltpu.VMEM((C_MAX, CHUNK), jnp.int32),
        pltpu.VMEM((C_MAX, CHUNK), jnp.int32),
        pltpu.VMEM((NBUF, CHUNK, D), jnp.float32),
        pltpu.VMEM_SHARED((N_PAD, D), jnp.float32),
        pltpu.SemaphoreType.DMA((NBUF,)),
        pltpu.SemaphoreType.DMA((NBUF,)),
    ]
    if with_count:
        out_type.append(jax.ShapeDtypeStruct((NC, N_PAD, CNT_W), jnp.float32))
        scratch.extend([
            pltpu.VMEM((CHUNK, CNT_W), jnp.float32),
            pltpu.VMEM_SHARED((N_PAD, CNT_W), jnp.float32),
            pltpu.SemaphoreType.DMA((NBUF,)),
        ])

    def body(src_h, dst_h, y_h, zrow_h, zcnt_h, ones_h, *rest):
        if with_count:
            (part_h, cnt_h, sidx, didx, rows, acc, sem_g, sem_s,
             ones, cacc, sem_c) = rest
        else:
            part_h, sidx, didx, rows, acc, sem_g, sem_s = rest
        c = lax.axis_index("c")
        s = lax.axis_index("s")
        n = lax.select(c == 0, C_SPLIT[0], C_SPLIT[1])
        base = lax.select(c == 0, s * C_SPLIT[0],
                          NS * C_SPLIT[0] + s * C_SPLIT[1])
        pltpu.sync_copy(src_h.at[pl.ds(base, C_MAX)], sidx)
        pltpu.sync_copy(dst_h.at[pl.ds(base, C_MAX)], didx)
        pltpu.sync_copy(zrow_h, acc.at[pl.ds(s * ROWS_PER_S, ROWS_PER_S)])
        if with_count:
            pltpu.sync_copy(zcnt_h, cacc.at[pl.ds(s * ROWS_PER_S, ROWS_PER_S)])
            pltpu.sync_copy(ones_h, ones)
        plsc.subcore_barrier()

        def gather(t):
            b = lax.rem(t, NBUF)
            return pltpu.make_async_copy(
                y_h.at[sidx.at[t]], rows.at[b], sem_g.at[b])

        def scat_start(t):
            b = lax.rem(t, NBUF)
            pltpu.async_copy(
                rows.at[b], acc.at[didx.at[t]], sem_s.at[b], add=True)

        def scat_wait(t):
            b = lax.rem(t, NBUF)
            pltpu.make_async_copy(
                rows.at[b], acc.at[didx.at[t]], sem_s.at[b]).wait()

        def cnt_start(t):
            b = lax.rem(t, NBUF)
            pltpu.async_copy(ones, cacc.at[didx.at[t]], sem_c.at[b], add=True)

        def cnt_wait(t):
            b = lax.rem(t, NBUF)
            pltpu.make_async_copy(ones, cacc.at[didx.at[t]], sem_c.at[b]).wait()

        for t0 in range(NBUF - 1):
            gather(t0).start()

        def step(t, carry):
            gather(t).wait()
            scat_start(t)
            if with_count:
                cnt_start(t)

            @pl.when(t + NBUF - 1 < n)
            def _():
                @pl.when(t >= 1)
                def _():
                    scat_wait(t - 1)
                    if with_count:
                        cnt_wait(t - 1)
                gather(t + NBUF - 1).start()

            return carry

        lax.fori_loop(0, n, step, 0)
        for k in range(NBUF):
            scat_wait(n - NBUF + k)
            if with_count:
                cnt_wait(n - NBUF + k)
        plsc.subcore_barrier()
        sl = pl.ds(s * ROWS_PER_S, ROWS_PER_S)
        pltpu.sync_copy(acc.at[sl], part_h.at[c, sl])
        if with_count:
            pltpu.sync_copy(cacc.at[sl], cnt_h.at[c, sl])

    return pl.kernel(
        body, mesh=mesh, out_type=out_type, scratch_types=scratch,
        compiler_params=pltpu.CompilerParams(use_tc_tiling_on_sc=False))


def _tc_layer1(x, wl_t, wr_t, b):
    def body(x_ref, wl_ref, wr_ref, b_ref, y_ref, r_ref):
        xv = x_ref[...]
        y_ref[...] = jnp.dot(xv, wl_ref[...], preferred_element_type=jnp.float32)
        r_ref[...] = (jnp.dot(xv, wr_ref[...], preferred_element_type=jnp.float32)
                      + b_ref[...])

    return pl.pallas_call(
        body,
        out_shape=(jax.ShapeDtypeStruct((N_NODES, HIDDEN), jnp.float32),
                   jax.ShapeDtypeStruct((N_NODES, HIDDEN), jnp.float32)),
    )(x, wl_t, wr_t, b)


def _tc_layer2(part, cnt, r1, wl_t, wr_t, b):
    def body(p_ref, c_ref, r1_ref, wl_ref, wr_ref, b_ref, y_ref, r_ref):
        psum = p_ref[0] + p_ref[1]
        cv = c_ref[0, :, 0:1] + c_ref[1, :, 0:1]
        mean = psum / jnp.maximum(cv, 1.0)
        h = jnp.maximum(mean[:N_NODES] + r1_ref[...], 0.0)
        y_ref[...] = jnp.dot(h, wl_ref[...], preferred_element_type=jnp.float32)
        r_ref[...] = (jnp.dot(h, wr_ref[...], preferred_element_type=jnp.float32)
                      + b_ref[...])

    return pl.pallas_call(
        body,
        out_shape=(jax.ShapeDtypeStruct((N_NODES, OUT_FEATS), jnp.float32),
                   jax.ShapeDtypeStruct((N_NODES, OUT_FEATS), jnp.float32)),
    )(part, cnt, r1, wl_t, wr_t, b)


def _tc_final(part, cnt, r2):
    def body(p_ref, c_ref, r2_ref, o_ref):
        psum = p_ref[0] + p_ref[1]
        cv = c_ref[0, :, 0:1] + c_ref[1, :, 0:1]
        mean = psum / jnp.maximum(cv, 1.0)
        o_ref[...] = mean[:N_NODES] + r2_ref[...]

    return pl.pallas_call(
        body,
        out_shape=jax.ShapeDtypeStruct((N_NODES, OUT_FEATS), jnp.float32),
    )(part, cnt, r2)


def kernel(x, edge_index, W1l, b1, W1r, W2l, b2, W2r):
    src = edge_index[0].astype(jnp.int32)
    dst = edge_index[1].astype(jnp.int32)
    pad = E_PAD - N_EDGES
    src_p = jnp.concatenate(
        [src, jnp.zeros((pad,), jnp.int32)]).reshape(-1, CHUNK)
    dst_p = jnp.concatenate(
        [dst, jnp.full((pad,), N_NODES, jnp.int32)]).reshape(-1, CHUNK)

    z_h = jnp.zeros((ROWS_PER_S, HIDDEN), jnp.float32)
    z_o = jnp.zeros((ROWS_PER_S, OUT_FEATS), jnp.float32)
    z_c = jnp.zeros((ROWS_PER_S, CNT_W), jnp.float32)
    ones_c = jnp.ones((CHUNK, CNT_W), jnp.float32)

    y1, r1 = _tc_layer1(x, W1l.T, W1r.T, b1.reshape(1, HIDDEN))
    part1, cnt = _sc_edge_agg(HIDDEN, True)(src_p, dst_p, y1, z_h, z_c, ones_c)
    y2, r2 = _tc_layer2(part1, cnt, r1, W2l.T, W2r.T, b2.reshape(1, OUT_FEATS))
    out2 = _sc_edge_agg(OUT_FEATS, False)(src_p, dst_p, y2, z_o, z_c, ones_c)
    part2 = out2[0] if isinstance(out2, (list, tuple)) else out2
    return _tc_final(part2, cnt, r2)


# speedup vs baseline: 1.0939x; 1.0939x over previous
import functools

import jax
import jax.numpy as jnp
from jax import lax
from jax.experimental import pallas as pl
from jax.experimental.pallas import tpu as pltpu
from jax.experimental.pallas import tpu_sc as plsc

N_NODES = 10000
N_EDGES = 320000
IN_FEATS = 128
HIDDEN = 64
OUT_FEATS = 32

NC = 2
NS = 16
CHUNK = 128
CHUNKS_PER_W = 79
E_PER_W = CHUNKS_PER_W * CHUNK
E_PAD = NC * NS * E_PER_W
N_PAD = 10240
ROWS_PER_S = N_PAD // NS
CNT_W = 8


NBUF = 3
C_SPLIT = (105, 53)
C_MAX = max(C_SPLIT)
assert sum(C_SPLIT) == 2 * CHUNKS_PER_W


def _sc_edge_agg(D, with_count):
    mesh = plsc.VectorSubcoreMesh(
        core_axis_name="c", subcore_axis_name="s",
        num_cores=NC, num_subcores=NS)

    out_type = [jax.ShapeDtypeStruct((NC, N_PAD, D), jnp.float32)]
    scratch = [
        pltpu.VMEM((C_MAX, CHUNK), jnp.int32),
        pltpu.VMEM((C_MAX, CHUNK), jnp.int32),
        pltpu.VMEM((NBUF, CHUNK, D), jnp.float32),
        pltpu.VMEM_SHARED((N_PAD, D), jnp.float32),
        pltpu.SemaphoreType.DMA((NBUF,)),
        pltpu.SemaphoreType.DMA((NBUF,)),
    ]
    if with_count:
        out_type.append(jax.ShapeDtypeStruct((NC, N_PAD, CNT_W), jnp.float32))
        scratch.extend([
            pltpu.VMEM((CHUNK, CNT_W), jnp.float32),
            pltpu.VMEM_SHARED((N_PAD, CNT_W), jnp.float32),
            pltpu.SemaphoreType.DMA((NBUF,)),
        ])

    def body(src_h, dst_h, y_h, zrow_h, zcnt_h, ones_h, *rest):
        if with_count:
            (part_h, cnt_h, sidx, didx, rows, acc, sem_g, sem_s,
             ones, cacc, sem_c) = rest
        else:
            part_h, sidx, didx, rows, acc, sem_g, sem_s = rest
        c = lax.axis_index("c")
        s = lax.axis_index("s")
        n = lax.select(c == 0, C_SPLIT[0], C_SPLIT[1])
        base = lax.select(c == 0, s * C_SPLIT[0],
                          NS * C_SPLIT[0] + s * C_SPLIT[1])
        pltpu.sync_copy(src_h.at[pl.ds(base, C_MAX)], sidx)
        pltpu.sync_copy(dst_h.at[pl.ds(base, C_MAX)], didx)
        pltpu.sync_copy(zrow_h, acc.at[pl.ds(s * ROWS_PER_S, ROWS_PER_S)])
        if with_count:
            pltpu.sync_copy(zcnt_h, cacc.at[pl.ds(s * ROWS_PER_S, ROWS_PER_S)])
            pltpu.sync_copy(ones_h, ones)
        plsc.subcore_barrier()

        def gather(t):
            b = lax.rem(t, NBUF)
            return pltpu.make_async_copy(
                y_h.at[sidx.at[t]], rows.at[b], sem_g.at[b])

        def scat_start(t):
            b = lax.rem(t, NBUF)
            pltpu.async_copy(
                rows.at[b], acc.at[didx.at[t]], sem_s.at[b], add=True)

        def scat_wait(t):
            b = lax.rem(t, NBUF)
            pltpu.make_async_copy(
                rows.at[b], acc.at[didx.at[t]], sem_s.at[b]).wait()

        def cnt_start(t):
            b = lax.rem(t, NBUF)
            pltpu.async_copy(ones, cacc.at[didx.at[t]], sem_c.at[b], add=True)

        def cnt_wait(t):
            b = lax.rem(t, NBUF)
            pltpu.make_async_copy(ones, cacc.at[didx.at[t]], sem_c.at[b]).wait()

        for t0 in range(NBUF - 1):
            gather(t0).start()

        def step(t, carry):
            gather(t).wait()
            scat_start(t)
            if with_count:
                cnt_start(t)

            @pl.when(t + NBUF - 1 < n)
            def _():
                @pl.when(t >= 1)
                def _():
                    scat_wait(t - 1)
                    if with_count:
                        cnt_wait(t - 1)
                gather(t + NBUF - 1).start()

            return carry

        lax.fori_loop(0, n, step, 0)
        for k in range(NBUF):
            scat_wait(n - NBUF + k)
            if with_count:
                cnt_wait(n - NBUF + k)
        plsc.subcore_barrier()
        sl = pl.ds(s * ROWS_PER_S, ROWS_PER_S)
        pltpu.sync_copy(acc.at[sl], part_h.at[c, sl])
        if with_count:
            pltpu.sync_copy(cacc.at[sl], cnt_h.at[c, sl])

    return pl.kernel(
        body, mesh=mesh, out_type=out_type, scratch_types=scratch,
        compiler_params=pltpu.CompilerParams(use_tc_tiling_on_sc=False))


def _tc_layer1(x, wl_t, wr_t, b):
    def body(x_ref, wl_ref, wr_ref, b_ref, y_ref, r_ref):
        xv = x_ref[...]
        y_ref[...] = jnp.dot(xv, wl_ref[...], preferred_element_type=jnp.float32)
        r_ref[...] = (jnp.dot(xv, wr_ref[...], preferred_element_type=jnp.float32)
                      + b_ref[...])

    return pl.pallas_call(
        body,
        out_shape=(jax.ShapeDtypeStruct((N_NODES, HIDDEN), jnp.float32),
                   jax.ShapeDtypeStruct((N_NODES, HIDDEN), jnp.float32)),
    )(x, wl_t, wr_t, b)


def _tc_layer2(part, cnt, r1, wl_t, wr_t, b):
    def body(p_ref, c_ref, r1_ref, wl_ref, wr_ref, b_ref, y_ref, r_ref):
        psum = p_ref[0] + p_ref[1]
        cv = c_ref[0, :, 0:1] + c_ref[1, :, 0:1]
        mean = psum / jnp.maximum(cv, 1.0)
        h = jnp.maximum(mean[:N_NODES] + r1_ref[...], 0.0)
        y_ref[...] = jnp.dot(h, wl_ref[...], preferred_element_type=jnp.float32)
        r_ref[...] = (jnp.dot(h, wr_ref[...], preferred_element_type=jnp.float32)
                      + b_ref[...])

    return pl.pallas_call(
        body,
        out_shape=(jax.ShapeDtypeStruct((N_NODES, OUT_FEATS), jnp.float32),
                   jax.ShapeDtypeStruct((N_NODES, OUT_FEATS), jnp.float32)),
    )(part, cnt, r1, wl_t, wr_t, b)


def _tc_final(part, cnt, r2):
    def body(p_ref, c_ref, r2_ref, o_ref):
        psum = p_ref[0] + p_ref[1]
        cv = c_ref[0, :, 0:1] + c_ref[1, :, 0:1]
        mean = psum / jnp.maximum(cv, 1.0)
        o_ref[...] = mean[:N_NODES] + r2_ref[...]

    return pl.pallas_call(
        body,
        out_shape=jax.ShapeDtypeStruct((N_NODES, OUT_FEATS), jnp.float32),
    )(part, cnt, r2)


def kernel(x, edge_index, W1l, b1, W1r, W2l, b2, W2r):
    src = edge_index[0].astype(jnp.int32)
    dst = edge_index[1].astype(jnp.int32)
    pad = E_PAD - N_EDGES
    src_p = jnp.concatenate(
        [src, jnp.zeros((pad,), jnp.int32)]).reshape(-1, CHUNK)
    dst_p = jnp.concatenate(
        [dst, jnp.full((pad,), N_NODES, jnp.int32)]).reshape(-1, CHUNK)

    z_h = jnp.zeros((ROWS_PER_S, HIDDEN), jnp.float32)
    z_o = jnp.zeros((ROWS_PER_S, OUT_FEATS), jnp.float32)
    z_c = jnp.zeros((ROWS_PER_S, CNT_W), jnp.float32)
    ones_c = jnp.ones((CHUNK, CNT_W), jnp.float32)

    y1, r1 = _tc_layer1(x, W1l.T, W1r.T, b1.reshape(1, HIDDEN))
    part1, cnt = _sc_edge_agg(HIDDEN, True)(src_p, dst_p, y1, z_h, z_c, ones_c)
    y2, r2 = _tc_layer2(part1, cnt, r1, W2l.T, W2r.T, b2.reshape(1, OUT_FEATS))
    out2 = _sc_edge_agg(OUT_FEATS, False)(src_p, dst_p, y2, z_o, z_c, ones_c)
    part2 = out2[0] if isinstance(out2, (list, tuple)) else out2
    return _tc_final(part2, cnt, r2)
